# baseline (speedup 1.0000x reference)
import functools

import jax
import jax.numpy as jnp
from jax.experimental import pallas as pl
from jax.experimental.pallas import tpu as pltpu


def _attn_block_body(x_ref, w1_ref, b1_ref, w2_ref, b2_ref, o_ref, *, inv_hw):
    x = x_ref[...]
    pooled = jnp.sum(x, axis=(2, 3), dtype=jnp.float32) * inv_hw
    h = jax.lax.dot_general(pooled, w1_ref[...],
                            (((1,), (1,)), ((), ())),
                            preferred_element_type=jnp.float32)
    h = jnp.maximum(h + b1_ref[...], 0.0)
    att = jax.lax.dot_general(h, w2_ref[...],
                              (((1,), (1,)), ((), ())),
                              preferred_element_type=jnp.float32)
    att = (att + b2_ref[...]).astype(o_ref.dtype)
    o_ref[...] = jnp.broadcast_to(att[:, :, None, None], o_ref.shape)


def _pick_row_block(n_rows, row_bytes, target_bytes):
    cap = max(1, target_bytes // row_bytes)
    tb = 1
    for d in range(1, n_rows + 1):
        if n_rows % d == 0 and d <= cap:
            tb = d
    return tb


def kernel(x_nchw, w1, b1, w2, b2):
    N, C, H, W = x_nchw.shape
    Cr = w1.shape[0]
    dtype = x_nchw.dtype
    itemsize = jnp.dtype(dtype).itemsize

    w1f = w1.astype(jnp.float32)
    w2f = w2.astype(jnp.float32)
    b1r = b1.reshape(1, Cr).astype(jnp.float32)
    b2r = b2.reshape(1, C).astype(jnp.float32)

    row_bytes = C * H * W * itemsize
    tb = _pick_row_block(N, row_bytes, target_bytes=2 * 1024 * 1024)
    nb = N // tb

    row_map = lambda i: (i, 0, 0, 0)
    fixed = lambda i: (0, 0)
    out = pl.pallas_call(
        functools.partial(_attn_block_body, inv_hw=1.0 / float(H * W)),
        out_shape=jax.ShapeDtypeStruct((N, C, H, W), dtype),
        grid=(nb,),
        in_specs=[
            pl.BlockSpec((tb, C, H, W), row_map),
            pl.BlockSpec((Cr, C), fixed),
            pl.BlockSpec((1, Cr), fixed),
            pl.BlockSpec((C, Cr), fixed),
            pl.BlockSpec((1, C), fixed),
        ],
        out_specs=pl.BlockSpec((tb, C, H, W), row_map),
        compiler_params=pltpu.CompilerParams(
            dimension_semantics=("arbitrary",),
            vmem_limit_bytes=48 * 1024 * 1024,
        ),
    )(x_nchw, w1f, b1r, w2f, b2r)
    return out


# speedup vs baseline: 11.3999x; 11.3999x over previous
import functools

import jax
import jax.numpy as jnp
from jax.experimental import pallas as pl
from jax.experimental.pallas import tpu as pltpu


def _attn_block_body(x_ref, w1_ref, b1_ref, w2_ref, b2_ref, o_ref, *, inv_hw):
    x = x_ref[...]
    pooled = jnp.sum(x, axis=(1, 2), dtype=jnp.float32) * inv_hw
    h = jax.lax.dot_general(pooled, w1_ref[...],
                            (((1,), (1,)), ((), ())),
                            preferred_element_type=jnp.float32)
    h = jnp.maximum(h + b1_ref[...], 0.0)
    att = jax.lax.dot_general(h, w2_ref[...],
                              (((1,), (1,)), ((), ())),
                              preferred_element_type=jnp.float32)
    att = (att + b2_ref[...]).astype(o_ref.dtype)
    o_ref[...] = jnp.broadcast_to(att[:, None, None, :], o_ref.shape)


def _pick_row_block(n_rows, row_bytes, target_bytes):
    cap = max(1, target_bytes // row_bytes)
    tb = 1
    for d in range(1, n_rows + 1):
        if n_rows % d == 0 and d <= cap:
            tb = d
    return tb


def kernel(x_nchw, w1, b1, w2, b2):
    N, C, H, W = x_nchw.shape
    Cr = w1.shape[0]
    dtype = x_nchw.dtype
    itemsize = jnp.dtype(dtype).itemsize

    x_t = jnp.transpose(x_nchw, (0, 2, 3, 1))

    w1f = w1.astype(jnp.float32)
    w2f = w2.astype(jnp.float32)
    b1r = b1.reshape(1, Cr).astype(jnp.float32)
    b2r = b2.reshape(1, C).astype(jnp.float32)

    row_bytes = C * H * W * itemsize
    tb = _pick_row_block(N, row_bytes, target_bytes=4 * 1024 * 1024)
    nb = N // tb

    row_map = lambda i: (i, 0, 0, 0)
    fixed = lambda i: (0, 0)
    out_t = pl.pallas_call(
        functools.partial(_attn_block_body, inv_hw=1.0 / float(H * W)),
        out_shape=jax.ShapeDtypeStruct((N, H, W, C), dtype),
        grid=(nb,),
        in_specs=[
            pl.BlockSpec((tb, H, W, C), row_map),
            pl.BlockSpec((Cr, C), fixed),
            pl.BlockSpec((1, Cr), fixed),
            pl.BlockSpec((C, Cr), fixed),
            pl.BlockSpec((1, C), fixed),
        ],
        out_specs=pl.BlockSpec((tb, H, W, C), row_map),
        compiler_params=pltpu.CompilerParams(
            dimension_semantics=("arbitrary",),
            vmem_limit_bytes=48 * 1024 * 1024,
        ),
    )(x_t, w1f, b1r, w2f, b2r)

    return jnp.transpose(out_t, (0, 3, 1, 2))


# speedup vs baseline: 11.9196x; 1.0456x over previous
import functools

import jax
import jax.numpy as jnp
from jax.experimental import pallas as pl
from jax.experimental.pallas import tpu as pltpu


def _attn_block_body(x_ref, w1_ref, b1_ref, w2_ref, b2_ref, o_ref, *, inv_hw):
    x = x_ref[...]
    pooled = jnp.sum(x, axis=(1, 2), dtype=jnp.float32) * inv_hw
    h = jax.lax.dot_general(pooled, w1_ref[...],
                            (((1,), (1,)), ((), ())),
                            preferred_element_type=jnp.float32)
    h = jnp.maximum(h + b1_ref[...], 0.0)
    att = jax.lax.dot_general(h, w2_ref[...],
                              (((1,), (1,)), ((), ())),
                              preferred_element_type=jnp.float32)
    att = (att + b2_ref[...]).astype(o_ref.dtype)
    o_ref[...] = jnp.broadcast_to(att[:, None, None, :], o_ref.shape)


def _pick_row_block(n_rows, row_bytes, target_bytes):
    cap = max(1, target_bytes // row_bytes)
    tb = 1
    for d in range(1, n_rows + 1):
        if n_rows % d == 0 and d <= cap:
            tb = d
    return tb


def kernel(x_nchw, w1, b1, w2, b2):
    N, C, H, W = x_nchw.shape
    Cr = w1.shape[0]
    dtype = x_nchw.dtype
    itemsize = jnp.dtype(dtype).itemsize

    x_t = jnp.transpose(x_nchw, (0, 2, 3, 1))

    w1f = w1.astype(jnp.float32)
    w2f = w2.astype(jnp.float32)
    b1r = b1.reshape(1, Cr).astype(jnp.float32)
    b2r = b2.reshape(1, C).astype(jnp.float32)

    row_bytes = C * H * W * itemsize
    tb = _pick_row_block(N, row_bytes, target_bytes=8 * 1024 * 1024)
    nb = N // tb

    row_map = lambda i: (i, 0, 0, 0)
    fixed = lambda i: (0, 0)
    out_t = pl.pallas_call(
        functools.partial(_attn_block_body, inv_hw=1.0 / float(H * W)),
        out_shape=jax.ShapeDtypeStruct((N, H, W, C), dtype),
        grid=(nb,),
        in_specs=[
            pl.BlockSpec((tb, H, W, C), row_map),
            pl.BlockSpec((Cr, C), fixed),
            pl.BlockSpec((1, Cr), fixed),
            pl.BlockSpec((C, Cr), fixed),
            pl.BlockSpec((1, C), fixed),
        ],
        out_specs=pl.BlockSpec((tb, H, W, C), row_map),
        compiler_params=pltpu.CompilerParams(
            dimension_semantics=("arbitrary",),
            vmem_limit_bytes=48 * 1024 * 1024,
        ),
    )(x_t, w1f, b1r, w2f, b2r)

    return jnp.transpose(out_t, (0, 3, 1, 2))
